# Initial kernel scaffold; baseline (speedup 1.0000x reference)
import jax
import jax.numpy as jnp
from jax.experimental import pallas as pl


def kernel(x, nlabel, edge_index, edge_weight, edge_mask, W_self_0, b_self_0, W_neigh_0, b_neigh_0, W_self_1, b_self_1, W_neigh_1, b_neigh_1, W_self_2, b_self_2, W_neigh_2, b_neigh_2, W_self_3, b_self_3, W_neigh_3, b_neigh_3, W_lin1, b_lin1, W_lin2, b_lin2):
    raise NotImplementedError("write your pallas kernel here")


# speedup vs baseline: 4.4536x; 4.4536x over previous
import functools

import jax
import jax.numpy as jnp
from jax import lax
from jax.experimental import pallas as pl
from jax.experimental.pallas import tpu as pltpu
from jax.experimental.pallas import tpu_sc as plsc

_NC, _NS = 2, 16
_NW = _NC * _NS
_LAT = 32
_CHUNK = 8000


def _sc_mesh():
    return plsc.VectorSubcoreMesh(core_axis_name="c", subcore_axis_name="s")


def _seg_sum_sc(hn_t, src, dst, w):
    n = hn_t.shape[1]
    e = src.shape[0]
    c = _CHUNK

    @functools.partial(
        pl.kernel,
        out_type=jax.ShapeDtypeStruct((_NW, n), jnp.float32),
        mesh=_sc_mesh(),
        compiler_params=pltpu.CompilerParams(needs_layout_passes=False),
        scratch_types=[
            pltpu.VMEM((n,), jnp.float32),
            pltpu.VMEM((n,), jnp.float32),
            pltpu.VMEM((c,), jnp.int32),
            pltpu.VMEM((c,), jnp.int32),
            pltpu.VMEM((c,), jnp.float32),
        ],
    )
    def k(hn_hbm, src_hbm, dst_hbm, w_hbm, out_hbm, col, acc, sb, db, wb):
        wid = lax.axis_index("s") * _NC + lax.axis_index("c")
        pltpu.sync_copy(hn_hbm.at[wid], col)

        def zero_body(i, carry):
            acc[pl.ds(i * 16, 16)] = jnp.zeros((16,), jnp.float32)
            return carry

        lax.fori_loop(0, n // 16, zero_body, 0)

        def chunk_body(ci, carry):
            off = pl.multiple_of(ci * c, c)
            pltpu.sync_copy(src_hbm.at[pl.ds(off, c)], sb)
            pltpu.sync_copy(dst_hbm.at[pl.ds(off, c)], db)
            pltpu.sync_copy(w_hbm.at[pl.ds(off, c)], wb)

            def inner(i, carry2):
                sv = sb[pl.ds(i * 16, 16)]
                dv = db[pl.ds(i * 16, 16)]
                wv = wb[pl.ds(i * 16, 16)]
                vals = plsc.load_gather(col, [sv]) * wv
                plsc.addupdate_scatter(acc, [dv], vals)
                return carry2

            lax.fori_loop(0, c // 16, inner, 0)
            return carry

        lax.fori_loop(0, e // c, chunk_body, 0)
        pltpu.sync_copy(acc, out_hbm.at[wid])

    return k(hn_t, src, dst, w)


def _deg_counts_sc(dst, n):
    e = dst.shape[0]
    ep = e // _NW

    @functools.partial(
        pl.kernel,
        out_type=jax.ShapeDtypeStruct((_NW, n), jnp.float32),
        mesh=_sc_mesh(),
        compiler_params=pltpu.CompilerParams(needs_layout_passes=False),
        scratch_types=[
            pltpu.VMEM((n,), jnp.float32),
            pltpu.VMEM((ep,), jnp.int32),
        ],
    )
    def k(dst_hbm, out_hbm, acc, db):
        wid = lax.axis_index("s") * _NC + lax.axis_index("c")

        def zero_body(i, carry):
            acc[pl.ds(i * 16, 16)] = jnp.zeros((16,), jnp.float32)
            return carry

        lax.fori_loop(0, n // 16, zero_body, 0)
        pltpu.sync_copy(dst_hbm.at[pl.ds(wid * ep, ep)], db)

        def inner(i, carry):
            dv = db[pl.ds(i * 16, 16)]
            plsc.addupdate_scatter(acc, [dv], jnp.full((16,), 1.0, jnp.float32))
            return carry

        lax.fori_loop(0, ep // 16, inner, 0)
        pltpu.sync_copy(acc, out_hbm.at[wid])

    return k(dst)


def _tc_pre_body(xt, wn, cnt, ew, em, hn_o, inv_o, w_o):
    hn_o[...] = jnp.dot(wn[...], xt[...], preferred_element_type=jnp.float32)
    deg = jnp.sum(cnt[...], axis=0, keepdims=True)
    inv_o[...] = 1.0 / jnp.maximum(deg, 1.0)
    w_o[...] = ew[...] * em[...]


def _tc_layer_body(hp, s, inv, ws, b, wnx, h_o, hn_o):
    h = jnp.tanh(jnp.dot(ws[...], hp[...], preferred_element_type=jnp.float32)
                 + b[...] + s[...] * inv[...])
    h_o[...] = h
    hn_o[...] = jnp.dot(wnx[...], h, preferred_element_type=jnp.float32)


def _tc_last_body(hp, s, inv, ws, b, h1, h2, h3, qm, cs_o):
    h4 = jnp.tanh(jnp.dot(ws[...], hp[...], preferred_element_type=jnp.float32)
                  + b[...] + s[...] * inv[...])
    cs_o[...] = jnp.concatenate([h1[...], h2[...], h3[...], h4], axis=0) * qm[...]


def _tc_head_body(ev, od, w1a, w1b, b1, w2, b2, o):
    z = jnp.maximum(
        jnp.dot(w1a[...], ev[...], preferred_element_type=jnp.float32)
        + jnp.dot(w1b[...], od[...], preferred_element_type=jnp.float32)
        + b1[...], 0.0)
    o[...] = jax.nn.sigmoid(jnp.dot(w2[...], z, preferred_element_type=jnp.float32)
                            + b2[...])


def _f32(shape):
    return jax.ShapeDtypeStruct(shape, jnp.float32)


def kernel(x, nlabel, edge_index, edge_weight, edge_mask,
           W_self_0, b_self_0, W_neigh_0, b_neigh_0,
           W_self_1, b_self_1, W_neigh_1, b_neigh_1,
           W_self_2, b_self_2, W_neigh_2, b_neigh_2,
           W_self_3, b_self_3, W_neigh_3, b_neigh_3,
           W_lin1, b_lin1, W_lin2, b_lin2):
    n = x.shape[0]
    e = edge_index.shape[1]
    src = edge_index[0]
    dst = edge_index[1]
    x_t = x.T
    ew_t = edge_weight.reshape(_NW, e // _NW)
    em_t = edge_mask.reshape(_NW, e // _NW)

    counts = _deg_counts_sc(dst, n)

    hn, invdeg, w_t = pl.pallas_call(
        _tc_pre_body,
        out_shape=[_f32((_LAT, n)), _f32((1, n)), _f32((_NW, e // _NW))],
    )(x_t, W_neigh_0, counts, ew_t, em_t)
    w = w_t.reshape(e)

    ws_list = [W_self_0, W_self_1, W_self_2, W_self_3]
    wn_next = [W_neigh_1, W_neigh_2, W_neigh_3]
    bsum = [(bs + bn).reshape(_LAT, 1) for bs, bn in
            [(b_self_0, b_neigh_0), (b_self_1, b_neigh_1),
             (b_self_2, b_neigh_2), (b_self_3, b_neigh_3)]]

    states = []
    hprev = x_t
    for l in range(3):
        s = _seg_sum_sc(hn, src, dst, w)
        hprev, hn = pl.pallas_call(
            _tc_layer_body,
            out_shape=[_f32((_LAT, n)), _f32((_LAT, n))],
        )(hprev, s, invdeg, ws_list[l], bsum[l], wn_next[l])
        states.append(hprev)

    s3 = _seg_sum_sc(hn, src, dst, w)
    qm = (nlabel[:, 0] == 1).astype(jnp.float32).reshape(1, n)
    cs = pl.pallas_call(
        _tc_last_body,
        out_shape=_f32((4 * _LAT, n)),
    )(hprev, s3, invdeg, ws_list[3], bsum[3], states[0], states[1], states[2], qm)

    csr = cs.reshape(4 * _LAT, n // 2, 2)
    even = csr[:, :, 0]
    odd = csr[:, :, 1]
    out = pl.pallas_call(
        _tc_head_body,
        out_shape=_f32((1, n // 2)),
    )(even, odd, W_lin1[:, :4 * _LAT], W_lin1[:, 4 * _LAT:],
      b_lin1.reshape(-1, 1), W_lin2, b_lin2.reshape(1, 1))
    return out.reshape(n // 2, 1)
